# baseline (speedup 1.0000x reference)
import functools

import jax
import jax.numpy as jnp
from jax import lax
from jax.experimental import pallas as pl
from jax.experimental.pallas import tpu as pltpu
from jax.experimental.pallas import tpu_sc as plsc

MIN_ROUTE_LEN = 2
MAX_ROUTE_LEN = 16

_ROWS = 16
_NBUF = 2


def _dense_kernel(dem_ref, tt_ref, hp_ref, trip_out_ref, scalars_ref):
    dem = dem_ref[...]
    tt = tt_ref[...]
    hp = hp_ref[...]

    hpf = (hp != 0).astype(jnp.float32)
    trip_times = tt * hpf
    trip_out_ref[...] = trip_times

    sd = dem * hpf
    ax = (1, 2)
    p_dt = jnp.sum(dem * trip_times, axis=ax)
    p_td = jnp.sum(dem, axis=ax)
    p_sv = jnp.sum(sd, axis=ax)

    vec = jnp.stack([p_dt, p_td, p_td - p_sv, p_sv], axis=1)
    scalars_ref[:, 0, :] = vec


def _sc_kernel(routes_hbm, nrl_hbm, hcr_hbm, dem_hbm, nt_hbm, hp_hbm,
               nsv_hbm, oob_hbm, tat_hbm,
               routes_v, pos_v, counts_v, nrl_v, hcr_v, oob_v, tat_v,
               dem_b0, dem_b1, nt_b0, nt_b1, hp_b0, hp_b1,
               sd0, sd1, sn0, sn1, sh0, sh1):
    R, L = routes_v.shape
    c = lax.axis_index("c")
    s = lax.axis_index("s")
    b = s * 2 + c

    n_chunks = dem_hbm.shape[1] // _ROWS
    dem_bufs = (dem_b0, dem_b1)
    nt_bufs = (nt_b0, nt_b1)
    hp_bufs = (hp_b0, hp_b1)
    dsems = (sd0, sd1)
    nsems = (sn0, sn1)
    hsems = (sh0, sh1)

    def _copies(chunk, k):
        r0 = chunk * _ROWS
        return (
            pltpu.make_async_copy(dem_hbm.at[b, pl.ds(r0, _ROWS)],
                                  dem_bufs[k], dsems[k]),
            pltpu.make_async_copy(nt_hbm.at[b, pl.ds(r0, _ROWS)],
                                  nt_bufs[k], nsems[k]),
            pltpu.make_async_copy(hp_hbm.at[b, pl.ds(r0, _ROWS)],
                                  hp_bufs[k], hsems[k]),
        )

    for k in range(_NBUF):
        for cp in _copies(k, k):
            cp.start()

    lanes = lax.iota(jnp.int32, 16)
    shifts = (lanes & 3) * 8
    word_of_lane = lanes >> 2
    zf = jnp.zeros((16,), jnp.float32)

    def chunk_body(i, accs, k):
        chunk = i * _NBUF + k
        for cp in _copies(chunk, k):
            cp.wait()
        a_tr, a_t0, a_t1, a_t2 = accs
        dem_b, nt_b, hp_b = dem_bufs[k], nt_bufs[k], hp_bufs[k]

        def row_body(r, accs2):
            c_tr, c_t0, c_t1, c_t2 = accs2
            rsplat = jnp.zeros((16,), jnp.int32) + r
            for q in range(32):
                dem_v = dem_b[r, pl.ds(q * 16, 16)]
                nt_v = nt_b[r, pl.ds(q * 16, 16)]
                g = plsc.load_gather(hp_b, [rsplat, q * 4 + word_of_lane])
                m = (g >> shifts) & 0xFF
                sd_v = jnp.where(m != 0, dem_v, zf)
                c_tr = c_tr + dem_v * nt_v.astype(jnp.float32)
                c_t0 = c_t0 + jnp.where(nt_v == 0, sd_v, zf)
                c_t1 = c_t1 + jnp.where(nt_v == 1, sd_v, zf)
                c_t2 = c_t2 + jnp.where(nt_v == 2, sd_v, zf)
            return (c_tr, c_t0, c_t1, c_t2)

        a_tr, a_t0, a_t1, a_t2 = lax.fori_loop(
            0, _ROWS, row_body, (a_tr, a_t0, a_t1, a_t2))

        @pl.when(chunk + _NBUF < n_chunks)
        def _next():
            for cp in _copies(chunk + _NBUF, k):
                cp.start()

        return (a_tr, a_t0, a_t1, a_t2)

    def ring_body(i, accs):
        for k in range(_NBUF):
            accs = chunk_body(i, accs, k)
        return accs

    a_tr, a_t0, a_t1, a_t2 = lax.fori_loop(
        0, n_chunks // _NBUF, ring_body, (zf, zf, zf, zf))

    tat_v[0, :] = zf + jnp.sum(a_tr)
    tat_v[1, :] = zf + jnp.sum(a_t0)
    tat_v[2, :] = zf + jnp.sum(a_t1)
    tat_v[3, :] = zf + jnp.sum(a_t2)
    pltpu.sync_copy(tat_v, tat_hbm.at[b])

    pltpu.sync_copy(routes_hbm.at[b], routes_v)
    pltpu.sync_copy(nrl_hbm, nrl_v)
    pltpu.sync_copy(hcr_hbm, hcr_v)

    accs = [jnp.zeros((16,), jnp.float32) for _ in range(R // 16)]
    oob_acc = jnp.zeros((16,), jnp.float32)
    for r in range(R):
        idx = routes_v[r, :]
        valid = idx > -1
        safe = jnp.where(valid, idx, 0)
        plsc.store_scatter(pos_v, [safe], lanes, mask=valid)
        g = plsc.load_gather(pos_v, [safe], mask=valid)
        first = jnp.logical_and(g == lanes, valid)
        cnt = plsc.all_reduce_population_count(first).astype(jnp.float32)
        rlen = plsc.all_reduce_population_count(valid)
        delta = jnp.maximum(MIN_ROUTE_LEN - rlen, 0)
        delta = jnp.where(rlen == 0, 0, delta)
        delta = delta + jnp.maximum(rlen - MAX_ROUTE_LEN, 0)
        oob_acc = oob_acc + delta.astype(jnp.float32)
        sel = lanes == (r % 16)
        k = r // 16
        accs[k] = jnp.where(sel, cnt, accs[k])
    for k in range(R // 16):
        counts_v[pl.ds(k * 16, 16)] = accs[k]
    pltpu.sync_copy(counts_v, nsv_hbm.at[b])

    bvec = jnp.full((16,), 0, jnp.int32) + b
    nrlb = plsc.load_gather(nrl_v, [bvec])
    hcrb = plsc.load_gather(hcr_v, [bvec])
    oob_v[...] = oob_acc + (nrlb - hcrb) * float(MIN_ROUTE_LEN)
    pltpu.sync_copy(oob_v, oob_hbm.at[b])


@jax.jit
def _run(demand, transit_times, n_transfers, has_path, batch_routes,
         nrl, hcr):
    B, N, _ = demand.shape
    _, R, L = batch_routes.shape
    hp8 = has_path.view(jnp.int8)
    G = 4
    bs_full = pl.BlockSpec((G, N, N), lambda g: (g, 0, 0))
    trip_times, scalars = pl.pallas_call(
        _dense_kernel,
        grid=(B // G,),
        in_specs=[bs_full, bs_full, bs_full],
        out_specs=[bs_full, pl.BlockSpec((G, 1, 4), lambda g: (g, 0, 0))],
        out_shape=[jax.ShapeDtypeStruct((B, N, N), jnp.float32),
                   jax.ShapeDtypeStruct((B, 1, 4), jnp.float32)],
    )(demand, transit_times, hp8)

    mesh = plsc.VectorSubcoreMesh(core_axis_name="c", subcore_axis_name="s",
                                  num_cores=2, num_subcores=16)
    nsv, oob, tat = pl.kernel(
        _sc_kernel,
        out_type=[jax.ShapeDtypeStruct((B, R), jnp.float32),
                  jax.ShapeDtypeStruct((B, 16), jnp.float32),
                  jax.ShapeDtypeStruct((B, 4, 16), jnp.float32)],
        mesh=mesh,
        scratch_types=[pltpu.VMEM((R, L), jnp.int32),
                       pltpu.VMEM((N,), jnp.int32),
                       pltpu.VMEM((R,), jnp.float32),
                       pltpu.VMEM((B,), jnp.float32),
                       pltpu.VMEM((B,), jnp.float32),
                       pltpu.VMEM((16,), jnp.float32),
                       pltpu.VMEM((4, 16), jnp.float32),
                       pltpu.VMEM((_ROWS, 512), jnp.float32),
                       pltpu.VMEM((_ROWS, 512), jnp.float32),
                       pltpu.VMEM((_ROWS, 512), jnp.int32),
                       pltpu.VMEM((_ROWS, 512), jnp.int32),
                       pltpu.VMEM((_ROWS, 128), jnp.int32),
                       pltpu.VMEM((_ROWS, 128), jnp.int32),
                       pltpu.SemaphoreType.DMA,
                       pltpu.SemaphoreType.DMA,
                       pltpu.SemaphoreType.DMA,
                       pltpu.SemaphoreType.DMA,
                       pltpu.SemaphoreType.DMA,
                       pltpu.SemaphoreType.DMA],
        compiler_params=pltpu.CompilerParams(needs_layout_passes=False),
    )(batch_routes, nrl, hcr,
      demand, n_transfers, has_path.view(jnp.int32))
    return trip_times, scalars, nsv, oob, tat


def kernel(demand, transit_times, total_route_time, n_routes_left_to_plan,
           n_transfers, has_path, batch_routes, has_current_route,
           n_disconnected):
    B = demand.shape[0]
    hcr = has_current_route.astype(jnp.float32)
    trip_times, scalars, nsv, oob, tat = _run(
        demand, transit_times, n_transfers, has_path, batch_routes,
        n_routes_left_to_plan, hcr)
    sc = scalars.reshape(B, 4)
    total_dmd_time = sc[:, 0]
    total_demand = sc[:, 1]
    unserved_demand = sc[:, 2]
    total_transfers = tat[:, 0, 0]
    t0 = tat[:, 1, 0]
    t1 = tat[:, 2, 0]
    t2 = tat[:, 3, 0]
    trips_at_transfers = jnp.stack(
        [t0, t1, t2, total_demand - t0 - t1 - t2], axis=1)
    n_stops_oob = oob[:, 0]
    n_stops_visited = nsv
    return (total_dmd_time, total_route_time, trips_at_transfers,
            total_demand, unserved_demand, total_transfers, trip_times,
            n_disconnected, n_stops_oob, n_stops_visited)


# speedup vs baseline: 4.9529x; 4.9529x over previous
import functools

import jax
import jax.numpy as jnp
from jax import lax
from jax.experimental import pallas as pl
from jax.experimental.pallas import tpu as pltpu
from jax.experimental.pallas import tpu_sc as plsc

MIN_ROUTE_LEN = 2
MAX_ROUTE_LEN = 16


def _dense_kernel(dem_ref, tt_ref, nt_ref, hp_ref, trip_out_ref, scalars_ref):
    dem = dem_ref[...]
    tt = tt_ref[...]
    nt = nt_ref[...]
    hp = hp_ref[...]

    hpf = (hp != 0).astype(jnp.float32)
    trip_times = tt * hpf
    trip_out_ref[...] = trip_times

    zero = jnp.zeros((), jnp.float32)
    sd = dem * hpf
    ax = (1, 2)
    p_dt = jnp.sum(dem * trip_times, axis=ax)
    p_td = jnp.sum(dem, axis=ax)
    p_sv = jnp.sum(sd, axis=ax)
    p_tr = jnp.sum(dem * nt.astype(jnp.float32), axis=ax)
    p_t0 = jnp.sum(jnp.where(nt == 0, sd, zero), axis=ax)
    p_t1 = jnp.sum(jnp.where(nt == 1, sd, zero), axis=ax)
    p_t2 = jnp.sum(jnp.where(nt == 2, sd, zero), axis=ax)

    vec = jnp.stack([p_dt, p_t0, p_t1, p_t2, p_td - p_t0 - p_t1 - p_t2,
                     p_td, p_td - p_sv, p_tr], axis=1)
    scalars_ref[:, 0, :] = vec


def _routes_sc_kernel(routes_hbm, nrl_hbm, hcr_hbm, nsv_hbm, oob_hbm,
                      routes_v, pos_v, counts_v, nrl_v, hcr_v, oob_v):
    R, L = routes_v.shape
    c = lax.axis_index("c")
    s = lax.axis_index("s")
    b = s * 2 + c

    pltpu.sync_copy(routes_hbm.at[b], routes_v)
    pltpu.sync_copy(nrl_hbm, nrl_v)
    pltpu.sync_copy(hcr_hbm, hcr_v)

    lanes = lax.iota(jnp.int32, 16)
    accs = [jnp.zeros((16,), jnp.float32) for _ in range(R // 16)]
    oob_acc = jnp.zeros((16,), jnp.float32)
    for r in range(R):
        idx = routes_v[r, :]
        valid = idx > -1
        safe = jnp.where(valid, idx, 0)
        plsc.store_scatter(pos_v, [safe], lanes, mask=valid)
        g = plsc.load_gather(pos_v, [safe], mask=valid)
        first = jnp.logical_and(g == lanes, valid)
        cnt = plsc.all_reduce_population_count(first).astype(jnp.float32)
        rlen = plsc.all_reduce_population_count(valid)
        delta = jnp.maximum(MIN_ROUTE_LEN - rlen, 0)
        delta = jnp.where(rlen == 0, 0, delta)
        delta = delta + jnp.maximum(rlen - MAX_ROUTE_LEN, 0)
        oob_acc = oob_acc + delta.astype(jnp.float32)
        sel = lanes == (r % 16)
        k = r // 16
        accs[k] = jnp.where(sel, cnt, accs[k])
    for k in range(R // 16):
        counts_v[pl.ds(k * 16, 16)] = accs[k]
    pltpu.sync_copy(counts_v, nsv_hbm.at[b])

    bvec = jnp.full((16,), 0, jnp.int32) + b
    nrlb = plsc.load_gather(nrl_v, [bvec])
    hcrb = plsc.load_gather(hcr_v, [bvec])
    oob_v[...] = oob_acc + (nrlb - hcrb) * float(MIN_ROUTE_LEN)
    pltpu.sync_copy(oob_v, oob_hbm.at[b])


@jax.jit
def _run(demand, transit_times, n_transfers, has_path, batch_routes,
         nrl, hcr):
    B, N, _ = demand.shape
    _, R, L = batch_routes.shape
    hp8 = has_path.view(jnp.int8)
    G = 4
    bs_full = pl.BlockSpec((G, N, N), lambda g: (g, 0, 0))
    trip_times, scalars = pl.pallas_call(
        _dense_kernel,
        grid=(B // G,),
        in_specs=[bs_full, bs_full, bs_full, bs_full],
        out_specs=[bs_full, pl.BlockSpec((G, 1, 8), lambda g: (g, 0, 0))],
        out_shape=[jax.ShapeDtypeStruct((B, N, N), jnp.float32),
                   jax.ShapeDtypeStruct((B, 1, 8), jnp.float32)],
    )(demand, transit_times, n_transfers, hp8)

    mesh = plsc.VectorSubcoreMesh(core_axis_name="c", subcore_axis_name="s",
                                  num_cores=2, num_subcores=16)
    nsv, oob = pl.kernel(
        _routes_sc_kernel,
        out_type=[jax.ShapeDtypeStruct((B, R), jnp.float32),
                  jax.ShapeDtypeStruct((B, 16), jnp.float32)],
        mesh=mesh,
        scratch_types=[pltpu.VMEM((R, L), jnp.int32),
                       pltpu.VMEM((N,), jnp.int32),
                       pltpu.VMEM((R,), jnp.float32),
                       pltpu.VMEM((B,), jnp.float32),
                       pltpu.VMEM((B,), jnp.float32),
                       pltpu.VMEM((16,), jnp.float32)],
        compiler_params=pltpu.CompilerParams(needs_layout_passes=False),
    )(batch_routes, nrl, hcr)
    return trip_times, scalars, nsv, oob


def kernel(demand, transit_times, total_route_time, n_routes_left_to_plan,
           n_transfers, has_path, batch_routes, has_current_route,
           n_disconnected):
    B = demand.shape[0]
    R = batch_routes.shape[1]
    hcr = has_current_route.astype(jnp.float32)
    trip_times, scalars, nsv, oob = _run(
        demand, transit_times, n_transfers, has_path, batch_routes,
        n_routes_left_to_plan, hcr)
    sc = scalars.reshape(B, 8)
    total_dmd_time = sc[:, 0]
    trips_at_transfers = sc[:, 1:5]
    total_demand = sc[:, 5]
    unserved_demand = sc[:, 6]
    total_transfers = sc[:, 7]
    n_stops_oob = oob[:, 0]
    n_stops_visited = nsv
    return (total_dmd_time, total_route_time, trips_at_transfers,
            total_demand, unserved_demand, total_transfers, trip_times,
            n_disconnected, n_stops_oob, n_stops_visited)
